# baseline (speedup 1.0000x reference)
import functools

import jax
import jax.numpy as jnp
from jax import lax
from jax.experimental import pallas as pl
from jax.experimental.pallas import tpu as pltpu
from jax.experimental.pallas import tpu_sc as plsc

N = 4096
B = 1024
NNZ = int(N * N * 0.1)

NW = 32
NLISTS = 64
T = 26624
PAD = NLISTS * T
NBINS = 16
CAP = 2048
STG = CAP + 128
BATCH = 512
NBATCH = T // BATCH
CHUNK_WORDS = 256 * N
TILE_WORDS = CHUNK_WORDS // 16

BN = 512


def _bin_body(packed_hbm, vals_hbm, idx_bins, val_bins,
              pb0, pb1, vb0, vb1, idx_st, val_st, semA, semB):
    wid = lax.axis_index("c") * 16 + lax.axis_index("s")
    zi = jnp.zeros((16,), jnp.int32)
    zf = jnp.zeros((16,), jnp.float32)
    lim = (NBATCH - 1) * BATCH

    def load(off, pb, vb, sem):
        pltpu.async_copy(packed_hbm.at[pl.ds(off, BATCH)], pb, sem)
        pltpu.async_copy(vals_hbm.at[pl.ds(off, BATCH)], vb, sem)

    def drain(pb, vb, sem):
        pltpu.make_async_copy(packed_hbm.at[pl.ds(0, BATCH)], pb, sem).wait()
        pltpu.make_async_copy(vals_hbm.at[pl.ds(0, BATCH)], vb, sem).wait()

    def process(pb, vb, cur):
        def vreg_body(k, cur):
            w = pb[pl.ds(k * 16, 16)]
            v = vb[pl.ds(k * 16, 16)]
            bin_ = lax.shift_right_logical(w, 20)
            lin = lax.bitwise_and(w, 0xFFFFF)
            ms = [bin_ == b for b in range(NBINS)]
            cnts = [plsc.all_reduce_population_count(m)[0] for m in ms]
            new = []
            for b in range(NBINS):
                cb = jnp.minimum(cur[b], CAP)
                plsc.store_compressed(idx_st.at[pl.ds(b * STG + cb, 16)],
                                      lin, mask=ms[b])
                plsc.store_compressed(val_st.at[pl.ds(b * STG + cb, 16)],
                                      v, mask=ms[b])
                new.append(cb + cnts[b])
            return tuple(new)
        return lax.fori_loop(0, BATCH // 16, vreg_body, cur)

    iota16 = lax.iota(jnp.int32, 16)
    for rnd in range(2):
        def zb(j, carry):
            idx_st[pl.ds(j * 16, 16)] = iota16 + j * 16
            val_st[pl.ds(j * 16, 16)] = zf
            return carry
        lax.fori_loop(0, NBINS * STG // 16, zb, 0)

        list_id = rnd * NW + wid
        base = list_id * T
        load(base, pb0, vb0, semA)

        def batch2(i, cur):
            load(base + (2 * i + 1) * BATCH, pb1, vb1, semB)
            drain(pb0, vb0, semA)
            cur = process(pb0, vb0, cur)
            load(base + jnp.minimum((2 * i + 2) * BATCH, lim), pb0, vb0, semA)
            drain(pb1, vb1, semB)
            cur = process(pb1, vb1, cur)
            return cur

        lax.fori_loop(0, NBATCH // 2, batch2,
                      tuple(jnp.int32(0) for _ in range(NBINS)))
        drain(pb0, vb0, semA)

        pltpu.sync_copy(idx_st, idx_bins.at[list_id])
        pltpu.sync_copy(val_st, val_bins.at[list_id])


def _accum_body(idx4, val4, zeros_hbm, wt_flat,
                shared, iv0, iv1, vv0, vv1, zsem, wsem, lsem0, lsem1,
                ssem0, ssem1):
    cid = lax.axis_index("c")
    sid = lax.axis_index("s")
    ivs, vvs = [iv0, iv1], [vv0, vv1]
    lsems, ssems = [lsem0, lsem1], [ssem0, ssem1]
    scat = [None, None]
    wo = None

    for p in range(NBINS // 2):
        chunk = p * 2 + cid
        if wo is not None:
            wo.wait()
        z = pltpu.async_copy(
            zeros_hbm.at[pl.ds(sid * TILE_WORDS, TILE_WORDS)],
            shared.at[pl.ds(sid * TILE_WORDS, TILE_WORDS)], zsem)
        lds = [None, None]
        first = sid * 4
        lds[0] = (pltpu.async_copy(idx4.at[first, chunk], ivs[0], lsem0),
                  pltpu.async_copy(val4.at[first, chunk], vvs[0], lsem0))
        z.wait()
        plsc.subcore_barrier()
        for li in range(4):
            b = li & 1
            nb = 1 - b
            if li < 3:
                if scat[nb] is not None:
                    scat[nb].wait()
                    scat[nb] = None
                lds[nb] = (
                    pltpu.async_copy(idx4.at[first + li + 1, chunk], ivs[nb],
                                     lsems[nb]),
                    pltpu.async_copy(val4.at[first + li + 1, chunk], vvs[nb],
                                     lsems[nb]))
            lds[b][0].wait()
            lds[b][1].wait()
            for j in range(STG // 128):
                pltpu.async_copy(vvs[b].at[j], shared.at[ivs[b].at[j]],
                                 ssems[b], add=True)
            scat[b] = pltpu.make_async_copy(idx4.at[0, 0], vvs[b], ssems[b])
        for b in range(2):
            if scat[b] is not None:
                scat[b].wait()
                scat[b] = None
        plsc.subcore_barrier()
        wo = pltpu.async_copy(
            shared.at[pl.ds(sid * TILE_WORDS, TILE_WORDS)],
            wt_flat.at[pl.ds(chunk * CHUNK_WORDS + sid * TILE_WORDS,
                             TILE_WORDS)], wsem)
    wo.wait()


_SC_MESH = plsc.VectorSubcoreMesh(core_axis_name="c", subcore_axis_name="s")
_SC_PARAMS = pltpu.CompilerParams(needs_layout_passes=False)

_bin_kernel = pl.kernel(
    _bin_body,
    out_type=(
        jax.ShapeDtypeStruct((NLISTS, NBINS * STG), jnp.int32),
        jax.ShapeDtypeStruct((NLISTS, NBINS * STG), jnp.float32),
    ),
    mesh=_SC_MESH,
    scratch_types=[
        pltpu.VMEM((BATCH,), jnp.int32),
        pltpu.VMEM((BATCH,), jnp.int32),
        pltpu.VMEM((BATCH,), jnp.float32),
        pltpu.VMEM((BATCH,), jnp.float32),
        pltpu.VMEM((NBINS * STG,), jnp.int32),
        pltpu.VMEM((NBINS * STG,), jnp.float32),
        pltpu.SemaphoreType.DMA,
        pltpu.SemaphoreType.DMA,
    ],
    compiler_params=_SC_PARAMS,
)

_accum_kernel = pl.kernel(
    _accum_body,
    out_type=jax.ShapeDtypeStruct((N * N,), jnp.float32),
    mesh=_SC_MESH,
    scratch_types=[
        pltpu.VMEM_SHARED((CHUNK_WORDS,), jnp.float32),
        pltpu.VMEM((STG // 128, 128), jnp.int32),
        pltpu.VMEM((STG // 128, 128), jnp.int32),
        pltpu.VMEM((STG // 128, 128), jnp.float32),
        pltpu.VMEM((STG // 128, 128), jnp.float32),
        pltpu.SemaphoreType.DMA,
        pltpu.SemaphoreType.DMA,
        pltpu.SemaphoreType.DMA,
        pltpu.SemaphoreType.DMA,
        pltpu.SemaphoreType.DMA,
        pltpu.SemaphoreType.DMA,
    ],
    compiler_params=_SC_PARAMS,
)


def _densify_wt(rows, cols, vals, zeros_page):
    packed = lax.shift_left(cols, 12) | rows
    packed_p = jnp.concatenate([packed, jnp.zeros((PAD - NNZ,), jnp.int32)])
    vals_p = jnp.concatenate([vals, jnp.zeros((PAD - NNZ,), jnp.float32)])
    idx_bins, val_bins = _bin_kernel(packed_p, vals_p)
    idx4 = idx_bins.reshape(NLISTS, NBINS, STG // 128, 128)
    val4 = val_bins.reshape(NLISTS, NBINS, STG // 128, 128)
    wt = _accum_kernel(idx4, val4, zeros_page)
    return wt.reshape(N, N)


def _layer_body(x_ref, w_ref, b_ref, o_ref, *, act):
    acc = lax.dot_general(
        x_ref[...], w_ref[...],
        dimension_numbers=(((1,), (0,)), ((), ())),
        preferred_element_type=jnp.float32,
        precision=lax.Precision.DEFAULT,
    )
    acc = acc + b_ref[...]
    if act:
        acc = jnp.tanh(acc)
    o_ref[...] = acc


def _layer(x, wt, bias, act):
    return pl.pallas_call(
        functools.partial(_layer_body, act=act),
        grid=(N // BN,),
        in_specs=[
            pl.BlockSpec((B, N), lambda j: (0, 0)),
            pl.BlockSpec((N, BN), lambda j: (0, j)),
            pl.BlockSpec((1, BN), lambda j: (0, j)),
        ],
        out_specs=pl.BlockSpec((B, BN), lambda j: (0, j)),
        out_shape=jax.ShapeDtypeStruct((B, N), jnp.float32),
    )(x, wt, bias.reshape(1, N))


def kernel(x, rows0, cols0, vals0, bias0, rows1, cols1, vals1, bias1,
           rows2, cols2, vals2, bias2):
    zeros_page = jnp.zeros((CHUNK_WORDS,), jnp.float32)
    wt0 = _densify_wt(rows0, cols0, vals0, zeros_page)
    wt1 = _densify_wt(rows1, cols1, vals1, zeros_page)
    wt2 = _densify_wt(rows2, cols2, vals2, zeros_page)
    h = _layer(x, wt0, bias0, act=True)
    h = _layer(h, wt1, bias1, act=True)
    return _layer(h, wt2, bias2, act=False)


# speedup vs baseline: 1.2892x; 1.2892x over previous
import functools

import jax
import jax.numpy as jnp
from jax import lax
from jax.experimental import pallas as pl
from jax.experimental.pallas import tpu as pltpu
from jax.experimental.pallas import tpu_sc as plsc

N = 4096
B = 1024
NNZ = int(N * N * 0.1)

NW = 32
NLISTS = 64
T = 26624
PAD = NLISTS * T
BATCH = 512
NBATCH = T // BATCH

NBINS = 10
BIG_ROWS = 410
STRIDE = 4096
CAP = 3200
BIN_OFF = tuple(b * STRIDE for b in range(NBINS))
BIN_CAP = (CAP,) * NBINS
STG_TOT = NBINS * STRIDE
CW_BIG = BIG_ROWS * N
TILE_BIG = CW_BIG // 16
CW_TAIL = 406 * N
TILE_TAIL = CW_TAIL // 16
SCAT_ROWS_BIG = 32
SCAT_ROWS_TAIL = 32

BN = 512


def _bin_body(packed_hbm, vals_hbm, idx_bins, val_bins,
              pb0, pb1, vb0, vb1, idx_st, val_st, semA, semB):
    wid = lax.axis_index("c") * 16 + lax.axis_index("s")
    zf = jnp.zeros((16,), jnp.float32)
    lim = (NBATCH - 1) * BATCH

    def load(off, pb, vb, sem):
        pltpu.async_copy(packed_hbm.at[pl.ds(off, BATCH)], pb, sem)
        pltpu.async_copy(vals_hbm.at[pl.ds(off, BATCH)], vb, sem)

    def drain(pb, vb, sem):
        pltpu.make_async_copy(packed_hbm.at[pl.ds(0, BATCH)], pb, sem).wait()
        pltpu.make_async_copy(vals_hbm.at[pl.ds(0, BATCH)], vb, sem).wait()

    def process(pb, vb, cur):
        def vreg_body(k, cur):
            w = pb[pl.ds(k * 16, 16)]
            v = vb[pl.ds(k * 16, 16)]
            c = lax.shift_right_logical(w, 12)
            bin_ = lax.shift_right_logical(c * 5116, 21)
            lin = w - bin_ * CW_BIG
            ms = [bin_ == b for b in range(NBINS)]
            cnts = [jnp.sum(m.astype(jnp.int32)) for m in ms]
            new = []
            for b in range(NBINS):
                cb = jnp.minimum(cur[b], BIN_CAP[b])
                plsc.store_compressed(idx_st.at[pl.ds(BIN_OFF[b] + cb, 16)],
                                      lin, mask=ms[b])
                plsc.store_compressed(val_st.at[pl.ds(BIN_OFF[b] + cb, 16)],
                                      v, mask=ms[b])
                new.append(cb + cnts[b])
            return tuple(new)
        return lax.fori_loop(0, BATCH // 16, vreg_body, cur)

    iota16 = lax.iota(jnp.int32, 16)
    for rnd in range(2):
        def zb(j, carry):
            idx_st[pl.ds(j * 16, 16)] = iota16 + j * 16
            val_st[pl.ds(j * 16, 16)] = zf
            return carry
        lax.fori_loop(0, STG_TOT // 16, zb, 0)

        list_id = rnd * NW + wid
        base = list_id * T
        load(base, pb0, vb0, semA)

        def batch2(i, cur):
            load(base + (2 * i + 1) * BATCH, pb1, vb1, semB)
            drain(pb0, vb0, semA)
            cur = process(pb0, vb0, cur)
            load(base + jnp.minimum((2 * i + 2) * BATCH, lim), pb0, vb0, semA)
            drain(pb1, vb1, semB)
            cur = process(pb1, vb1, cur)
            return cur

        lax.fori_loop(0, NBATCH // 2, batch2,
                      tuple(jnp.int32(0) for _ in range(NBINS)))
        drain(pb0, vb0, semA)

        pltpu.sync_copy(idx_st, idx_bins.at[list_id])
        pltpu.sync_copy(val_st, val_bins.at[list_id])


def _accum_body(idx3, val3, zeros_hbm, wt_flat,
                shared, iv0, iv1, vv0, vv1, zsem, wsem, lsem0, lsem1,
                ssem0, ssem1):
    cid = lax.axis_index("c")
    sid = lax.axis_index("s")
    ivs, vvs = [iv0, iv1], [vv0, vv1]
    lsems, ssems = [lsem0, lsem1], [ssem0, ssem1]
    first = sid * 4

    def iv_sl(b, nrows):
        return ivs[b] if nrows == SCAT_ROWS_BIG else ivs[b].at[pl.ds(0, nrows)]

    def vv_sl(b, nrows):
        return vvs[b] if nrows == SCAT_ROWS_BIG else vvs[b].at[pl.ds(0, nrows)]

    def do_chunk(row0, nrows, tile_words, out_base, wo_prev):
        if wo_prev is not None:
            wo_prev.wait()
        z = pltpu.async_copy(
            zeros_hbm.at[pl.ds(sid * tile_words, tile_words)],
            shared.at[pl.ds(sid * tile_words, tile_words)], zsem)

        def load(li, b):
            return (pltpu.async_copy(
                        idx3.at[first + li].at[pl.ds(row0, nrows)],
                        iv_sl(b, nrows), lsems[b]),
                    pltpu.async_copy(
                        val3.at[first + li].at[pl.ds(row0, nrows)],
                        vv_sl(b, nrows), lsems[b]))

        lds = [load(0, 0), None]
        scat = [None, None]
        z.wait()
        plsc.subcore_barrier()
        for li in range(4):
            b = li & 1
            nb = 1 - b
            if li < 3:
                if scat[nb] is not None:
                    scat[nb].wait()
                    scat[nb] = None
                lds[nb] = load(li + 1, nb)
            lds[b][0].wait()
            lds[b][1].wait()
            for j in range(nrows):
                pltpu.async_copy(vvs[b].at[j], shared.at[ivs[b].at[j]],
                                 ssems[b], add=True)
            scat[b] = pltpu.make_async_copy(
                val3.at[0].at[pl.ds(row0, nrows)], vv_sl(b, nrows), ssems[b])
        for b in range(2):
            if scat[b] is not None:
                scat[b].wait()
        plsc.subcore_barrier()
        return pltpu.async_copy(
            shared.at[pl.ds(sid * tile_words, tile_words)],
            wt_flat.at[pl.ds(out_base + sid * tile_words, tile_words)], wsem)

    wo = None
    for p in range(4):
        chunk = p * 2 + cid
        wo = do_chunk(chunk * (STRIDE // 128), SCAT_ROWS_BIG, TILE_BIG,
                      chunk * CW_BIG, wo)
    wo.wait()

    @pl.when(cid == 0)
    def _big8():
        wo2 = do_chunk(8 * (STRIDE // 128), SCAT_ROWS_BIG, TILE_BIG,
                       8 * CW_BIG, None)
        wo2.wait()

    @pl.when(cid == 1)
    def _tail9():
        wo3 = do_chunk(9 * (STRIDE // 128), SCAT_ROWS_TAIL, TILE_TAIL,
                       9 * CW_BIG, None)
        wo3.wait()


_SC_MESH = plsc.VectorSubcoreMesh(core_axis_name="c", subcore_axis_name="s")
_SC_PARAMS = pltpu.CompilerParams(needs_layout_passes=False)

_bin_kernel = pl.kernel(
    _bin_body,
    out_type=(
        jax.ShapeDtypeStruct((NLISTS, STG_TOT), jnp.int32),
        jax.ShapeDtypeStruct((NLISTS, STG_TOT), jnp.float32),
    ),
    mesh=_SC_MESH,
    scratch_types=[
        pltpu.VMEM((BATCH,), jnp.int32),
        pltpu.VMEM((BATCH,), jnp.int32),
        pltpu.VMEM((BATCH,), jnp.float32),
        pltpu.VMEM((BATCH,), jnp.float32),
        pltpu.VMEM((STG_TOT,), jnp.int32),
        pltpu.VMEM((STG_TOT,), jnp.float32),
        pltpu.SemaphoreType.DMA,
        pltpu.SemaphoreType.DMA,
    ],
    compiler_params=_SC_PARAMS,
)

_accum_kernel = pl.kernel(
    _accum_body,
    out_type=jax.ShapeDtypeStruct((N * N,), jnp.float32),
    mesh=_SC_MESH,
    scratch_types=[
        pltpu.VMEM_SHARED((CW_BIG,), jnp.float32),
        pltpu.VMEM((SCAT_ROWS_BIG, 128), jnp.int32),
        pltpu.VMEM((SCAT_ROWS_BIG, 128), jnp.int32),
        pltpu.VMEM((SCAT_ROWS_BIG, 128), jnp.float32),
        pltpu.VMEM((SCAT_ROWS_BIG, 128), jnp.float32),
        pltpu.SemaphoreType.DMA,
        pltpu.SemaphoreType.DMA,
        pltpu.SemaphoreType.DMA,
        pltpu.SemaphoreType.DMA,
        pltpu.SemaphoreType.DMA,
        pltpu.SemaphoreType.DMA,
    ],
    compiler_params=_SC_PARAMS,
)


def _densify_wt(rows, cols, vals, zeros_page):
    packed = lax.shift_left(cols, 12) | rows
    packed_p = jnp.concatenate([packed, jnp.zeros((PAD - NNZ,), jnp.int32)])
    vals_p = jnp.concatenate([vals, jnp.zeros((PAD - NNZ,), jnp.float32)])
    idx_bins, val_bins = _bin_kernel(packed_p, vals_p)
    idx3 = idx_bins.reshape(NLISTS, STG_TOT // 128, 128)
    val3 = val_bins.reshape(NLISTS, STG_TOT // 128, 128)
    wt = _accum_kernel(idx3, val3, zeros_page)
    return wt.reshape(N, N)


def _layer_body(x_ref, w_ref, b_ref, o_ref, *, act):
    acc = lax.dot_general(
        x_ref[...], w_ref[...],
        dimension_numbers=(((1,), (0,)), ((), ())),
        preferred_element_type=jnp.float32,
        precision=lax.Precision.DEFAULT,
    )
    acc = acc + b_ref[...]
    if act:
        acc = jnp.tanh(acc)
    o_ref[...] = acc


def _layer(x, wt, bias, act):
    return pl.pallas_call(
        functools.partial(_layer_body, act=act),
        grid=(N // BN,),
        in_specs=[
            pl.BlockSpec((B, N), lambda j: (0, 0)),
            pl.BlockSpec((N, BN), lambda j: (0, j)),
            pl.BlockSpec((1, BN), lambda j: (0, j)),
        ],
        out_specs=pl.BlockSpec((B, BN), lambda j: (0, j)),
        out_shape=jax.ShapeDtypeStruct((B, N), jnp.float32),
    )(x, wt, bias.reshape(1, N))


def kernel(x, rows0, cols0, vals0, bias0, rows1, cols1, vals1, bias1,
           rows2, cols2, vals2, bias2):
    zeros_page = jnp.zeros((CW_BIG,), jnp.float32)
    wt0 = _densify_wt(rows0, cols0, vals0, zeros_page)
    wt1 = _densify_wt(rows1, cols1, vals1, zeros_page)
    wt2 = _densify_wt(rows2, cols2, vals2, zeros_page)
    h = _layer(x, wt0, bias0, act=True)
    h = _layer(h, wt1, bias1, act=True)
    return _layer(h, wt2, bias2, act=False)
